# baseline (speedup 1.0000x reference)
import jax
import jax.numpy as jnp
from jax.experimental import pallas as pl
from jax.experimental.pallas import tpu as pltpu


def _make_body(H, W, C, BATCH):
    HW = H * W

    def body(x_ref, mask_ref, w1s_ref, w2e_ref, b1_ref, b2c_ref, o_ref):
        colmasks = {-1: mask_ref[0], 1: mask_ref[1]}

        def coltaps(vb):
            zs = []
            for dw in (-1, 0, 1):
                if dw == 0:
                    zs.append(vb)
                else:
                    d = dw % HW
                    zs.append(jnp.concatenate([vb[:, d:], vb[:, :d]],
                                              axis=1) * colmasks[dw])
            return zs

        cw = 896
        halves = tuple(
            (s, min(s + cw, HW), max(0, s - 128), min(HW, s + cw + 128))
            for s in range(0, HW, cw))

        def rowsum_win(y, s, e, ws, we):
            w0, width = s - ws, e - s
            y0 = y[C:2 * C, w0:w0 + width]
            om = w0 - W
            if om >= 0:
                down = y[:C, om:om + width]
            else:
                down = jnp.concatenate(
                    [jnp.zeros((C, -om), y.dtype), y[:C, :width + om]], axis=1)
            op = w0 + W
            if op + width <= we - ws:
                up = y[2 * C:, op:op + width]
            else:
                pad = op + width - (we - ws)
                up = jnp.concatenate(
                    [y[2 * C:, op:op + width - pad],
                     jnp.zeros((C, pad), y.dtype)], axis=1)
            return y0 + up + down

        out1s = []
        for b in range(BATCH):
            xb = x_ref[b].astype(jnp.bfloat16)
            zs = coltaps(xb)
            parts = []
            for s, e, ws, we in halves:
                z = jnp.concatenate([zz[:, ws:we] for zz in zs], axis=0)
                y = jnp.dot(w1s_ref[...], z,
                            preferred_element_type=jnp.float32)
                parts.append(jnp.maximum(
                    rowsum_win(y, s, e, ws, we) + b1_ref[...], 0.0))
            out1s.append(jnp.concatenate(parts, axis=1))

        for b in range(BATCH):
            xb = x_ref[b].astype(jnp.bfloat16)
            zs = coltaps(out1s[b].astype(jnp.bfloat16)) + [xb]
            for s, e, ws, we in halves:
                z2 = jnp.concatenate([zz[:, ws:we] for zz in zs], axis=0)
                y2 = jnp.dot(w2e_ref[...], z2,
                             preferred_element_type=jnp.float32)
                o_ref[b, :, s:e] = jnp.maximum(
                    rowsum_win(y2, s, e, ws, we) + b2c_ref[...], 0.0)

    return body


def kernel(x, w1_hwio, w2_hwio, ws_hwio, bn1_scale, bn1_bias,
           bn2_scale, bn2_bias, bns_scale, bns_bias):
    N, Cin, H, W = x.shape
    HW = H * W
    C = bn1_scale.shape[-1]

    Cin_p = -(-Cin // 8) * 8
    xr = x.reshape(N, Cin, HW).astype(jnp.float32)
    if Cin_p != Cin:
        xr = jnp.pad(xr, ((0, 0), (0, Cin_p - Cin), (0, 0)))

    cols = jnp.broadcast_to(jnp.arange(W).reshape(1, W), (H, W))
    tap_mask = jnp.stack(
        [((cols + dw >= 0) & (cols + dw < W)).reshape(1, HW)
         for dw in (-1, 1)], axis=0).astype(jnp.bfloat16)

    def prep3x3(w_hwio, scale, ci_pad):
        w = w_hwio * scale
        ci = w.shape[2]
        if ci_pad != ci:
            w = jnp.pad(w, ((0, 0), (0, 0), (0, ci_pad - ci), (0, 0)))
        return jnp.transpose(w, (3, 0, 1, 2)).reshape(C, 9 * ci_pad)

    w1 = prep3x3(w1_hwio, bn1_scale, Cin_p)
    w2 = prep3x3(w2_hwio, bn2_scale, C)

    ws = ws_hwio[0, 0] * bns_scale
    if Cin_p != Cin:
        ws = jnp.pad(ws, ((0, Cin_p - Cin), (0, 0)))
    ws = ws.T

    w1s = jnp.concatenate(
        [w1[:, :3 * Cin_p], w1[:, 3 * Cin_p:6 * Cin_p], w1[:, 6 * Cin_p:]],
        axis=0).astype(jnp.bfloat16)
    w2s = jnp.concatenate(
        [w2[:, :3 * C], w2[:, 3 * C:6 * C], w2[:, 6 * C:]], axis=0)
    sc_col = jnp.zeros((3 * C, Cin_p), jnp.float32).at[C:2 * C].set(ws)
    w2e = jnp.concatenate([w2s, sc_col], axis=1).astype(jnp.bfloat16)

    b1 = bn1_bias.reshape(C, 1).astype(jnp.float32)
    b2c = (bn2_bias + bns_bias).reshape(C, 1).astype(jnp.float32)

    def const_spec(shape):
        return pl.BlockSpec(shape, lambda n: (0,) * len(shape))

    BATCH = 2 if N % 2 == 0 else 1
    flops = 2 * N * HW * C * (9 * Cin_p + 9 * C + Cin_p)
    bytes_accessed = (xr.size * 4 + tap_mask.size * 2 + w1s.size * 2 +
                      w2e.size * 2 + 2 * C * 4 + N * C * HW * 4)
    cost = pl.CostEstimate(flops=flops, transcendentals=0,
                           bytes_accessed=bytes_accessed)

    out = pl.pallas_call(
        _make_body(H, W, C, BATCH),
        out_shape=jax.ShapeDtypeStruct((N, C, HW), jnp.float32),
        grid=(N // BATCH,),
        in_specs=[
            pl.BlockSpec((BATCH, Cin_p, HW), lambda n: (n, 0, 0)),
            const_spec((2, 1, HW)),
            const_spec((3 * C, 3 * Cin_p)),
            const_spec((3 * C, 3 * C + Cin_p)),
            const_spec((C, 1)),
            const_spec((C, 1)),
        ],
        out_specs=pl.BlockSpec((BATCH, C, HW), lambda n: (n, 0, 0)),
        compiler_params=pltpu.CompilerParams(
            dimension_semantics=("parallel",)),
        cost_estimate=cost,
    )(xr, tap_mask, w1s, w2e, b1, b2c)

    return out.reshape(N, C, H, W)


# speedup vs baseline: 1.0521x; 1.0521x over previous
import jax
import jax.numpy as jnp
from jax.experimental import pallas as pl
from jax.experimental.pallas import tpu as pltpu


def _make_body(H, W, C, BATCH):
    HW = H * W

    def body(x_ref, mask_ref, w1s_ref, w2e_ref, b1_ref, b2c_ref, o_ref):
        colmasks = {-1: mask_ref[0], 1: mask_ref[1]}

        def coltaps(vb):
            zs = []
            for dw in (-1, 0, 1):
                if dw == 0:
                    zs.append(vb)
                else:
                    d = dw % HW
                    zs.append(jnp.concatenate([vb[:, d:], vb[:, :d]],
                                              axis=1) * colmasks[dw])
            return zs

        cw = 896
        halves = tuple(
            (s, min(s + cw, HW), max(0, s - 128), min(HW, s + cw + 128))
            for s in range(0, HW, cw))

        def rowsum_win(y, s, e, ws, we):
            w0, width = s - ws, e - s
            y0 = y[C:2 * C, w0:w0 + width]
            ym = y[:C].astype(jnp.bfloat16)
            yp = y[2 * C:].astype(jnp.bfloat16)
            om = w0 - W
            if om >= 0:
                down = ym[:, om:om + width]
            else:
                down = jnp.concatenate(
                    [jnp.zeros((C, -om), ym.dtype), ym[:, :width + om]], axis=1)
            op = w0 + W
            if op + width <= we - ws:
                up = yp[:, op:op + width]
            else:
                pad = op + width - (we - ws)
                up = jnp.concatenate(
                    [yp[:, op:op + width - pad],
                     jnp.zeros((C, pad), ym.dtype)], axis=1)
            return y0 + (up + down).astype(y.dtype)

        out1s = []
        for b in range(BATCH):
            xb = x_ref[b].astype(jnp.bfloat16)
            zs = coltaps(xb)
            parts = []
            for s, e, ws, we in halves:
                z = jnp.concatenate([zz[:, ws:we] for zz in zs], axis=0)
                y = jnp.dot(w1s_ref[...], z,
                            preferred_element_type=jnp.float32)
                parts.append(jnp.maximum(
                    rowsum_win(y, s, e, ws, we) + b1_ref[...],
                    0.0).astype(jnp.bfloat16))
            out1s.append(jnp.concatenate(parts, axis=1))

        for b in range(BATCH):
            xb = x_ref[b].astype(jnp.bfloat16)
            zs = coltaps(out1s[b]) + [xb]
            for s, e, ws, we in halves:
                z2 = jnp.concatenate([zz[:, ws:we] for zz in zs], axis=0)
                y2 = jnp.dot(w2e_ref[...], z2,
                             preferred_element_type=jnp.float32)
                o_ref[b, :, s:e] = jnp.maximum(
                    rowsum_win(y2, s, e, ws, we) + b2c_ref[...], 0.0)

    return body


def kernel(x, w1_hwio, w2_hwio, ws_hwio, bn1_scale, bn1_bias,
           bn2_scale, bn2_bias, bns_scale, bns_bias):
    N, Cin, H, W = x.shape
    HW = H * W
    C = bn1_scale.shape[-1]

    Cin_p = -(-Cin // 8) * 8
    xr = x.reshape(N, Cin, HW).astype(jnp.float32)
    if Cin_p != Cin:
        xr = jnp.pad(xr, ((0, 0), (0, Cin_p - Cin), (0, 0)))

    cols = jnp.broadcast_to(jnp.arange(W).reshape(1, W), (H, W))
    tap_mask = jnp.stack(
        [((cols + dw >= 0) & (cols + dw < W)).reshape(1, HW)
         for dw in (-1, 1)], axis=0).astype(jnp.bfloat16)

    def prep3x3(w_hwio, scale, ci_pad):
        w = w_hwio * scale
        ci = w.shape[2]
        if ci_pad != ci:
            w = jnp.pad(w, ((0, 0), (0, 0), (0, ci_pad - ci), (0, 0)))
        return jnp.transpose(w, (3, 0, 1, 2)).reshape(C, 9 * ci_pad)

    w1 = prep3x3(w1_hwio, bn1_scale, Cin_p)
    w2 = prep3x3(w2_hwio, bn2_scale, C)

    ws = ws_hwio[0, 0] * bns_scale
    if Cin_p != Cin:
        ws = jnp.pad(ws, ((0, Cin_p - Cin), (0, 0)))
    ws = ws.T

    w1s = jnp.concatenate(
        [w1[:, :3 * Cin_p], w1[:, 3 * Cin_p:6 * Cin_p], w1[:, 6 * Cin_p:]],
        axis=0).astype(jnp.bfloat16)
    w2s = jnp.concatenate(
        [w2[:, :3 * C], w2[:, 3 * C:6 * C], w2[:, 6 * C:]], axis=0)
    sc_col = jnp.zeros((3 * C, Cin_p), jnp.float32).at[C:2 * C].set(ws)
    w2e = jnp.concatenate([w2s, sc_col], axis=1).astype(jnp.bfloat16)

    b1 = bn1_bias.reshape(C, 1).astype(jnp.float32)
    b2c = (bn2_bias + bns_bias).reshape(C, 1).astype(jnp.float32)

    def const_spec(shape):
        return pl.BlockSpec(shape, lambda n: (0,) * len(shape))

    BATCH = 2 if N % 2 == 0 else 1
    flops = 2 * N * HW * C * (9 * Cin_p + 9 * C + Cin_p)
    bytes_accessed = (xr.size * 4 + tap_mask.size * 2 + w1s.size * 2 +
                      w2e.size * 2 + 2 * C * 4 + N * C * HW * 4)
    cost = pl.CostEstimate(flops=flops, transcendentals=0,
                           bytes_accessed=bytes_accessed)

    out = pl.pallas_call(
        _make_body(H, W, C, BATCH),
        out_shape=jax.ShapeDtypeStruct((N, C, HW), jnp.float32),
        grid=(N // BATCH,),
        in_specs=[
            pl.BlockSpec((BATCH, Cin_p, HW), lambda n: (n, 0, 0)),
            const_spec((2, 1, HW)),
            const_spec((3 * C, 3 * Cin_p)),
            const_spec((3 * C, 3 * C + Cin_p)),
            const_spec((C, 1)),
            const_spec((C, 1)),
        ],
        out_specs=pl.BlockSpec((BATCH, C, HW), lambda n: (n, 0, 0)),
        compiler_params=pltpu.CompilerParams(
            dimension_semantics=("parallel",)),
        cost_estimate=cost,
    )(xr, tap_mask, w1s, w2e, b1, b2c)

    return out.reshape(N, C, H, W)


# speedup vs baseline: 1.0744x; 1.0212x over previous
import jax
import jax.numpy as jnp
from jax.experimental import pallas as pl
from jax.experimental.pallas import tpu as pltpu


def _make_body(H, W, C, BATCH):
    HW = H * W

    def body(x_ref, mask_ref, w1s_ref, w2e_ref, b1_ref, b2c_ref, o_ref):
        colmasks = {-1: mask_ref[0], 1: mask_ref[1]}

        def coltaps(vb):
            zs = []
            for dw in (-1, 0, 1):
                if dw == 0:
                    zs.append(vb)
                else:
                    d = dw % HW
                    zs.append(jnp.concatenate([vb[:, d:], vb[:, :d]],
                                              axis=1) * colmasks[dw])
            return zs

        cw = 1664
        halves = tuple(
            (s, min(s + cw, HW), max(0, s - 128), min(HW, s + cw + 128))
            for s in range(0, HW, cw))

        def rowsum_win(y, s, e, ws, we):
            w0, width = s - ws, e - s
            y0 = y[C:2 * C, w0:w0 + width]
            ym = y[:C].astype(jnp.bfloat16)
            yp = y[2 * C:].astype(jnp.bfloat16)
            om = w0 - W
            if om >= 0:
                down = ym[:, om:om + width]
            else:
                down = jnp.concatenate(
                    [jnp.zeros((C, -om), ym.dtype), ym[:, :width + om]], axis=1)
            op = w0 + W
            if op + width <= we - ws:
                up = yp[:, op:op + width]
            else:
                pad = op + width - (we - ws)
                up = jnp.concatenate(
                    [yp[:, op:op + width - pad],
                     jnp.zeros((C, pad), ym.dtype)], axis=1)
            return y0 + (up + down).astype(y.dtype)

        out1s = []
        for b in range(BATCH):
            xb = x_ref[b].astype(jnp.bfloat16)
            zs = coltaps(xb)
            parts = []
            for s, e, ws, we in halves:
                z = jnp.concatenate([zz[:, ws:we] for zz in zs], axis=0)
                y = jnp.dot(w1s_ref[...], z,
                            preferred_element_type=jnp.float32)
                parts.append(jnp.maximum(
                    rowsum_win(y, s, e, ws, we) + b1_ref[...],
                    0.0).astype(jnp.bfloat16))
            out1s.append(jnp.concatenate(parts, axis=1))

        for b in range(BATCH):
            xb = x_ref[b].astype(jnp.bfloat16)
            zs = coltaps(out1s[b]) + [xb]
            for s, e, ws, we in halves:
                z2 = jnp.concatenate([zz[:, ws:we] for zz in zs], axis=0)
                y2 = jnp.dot(w2e_ref[...], z2,
                             preferred_element_type=jnp.float32)
                o_ref[b, :, s:e] = jnp.maximum(
                    rowsum_win(y2, s, e, ws, we) + b2c_ref[...], 0.0)

    return body


def kernel(x, w1_hwio, w2_hwio, ws_hwio, bn1_scale, bn1_bias,
           bn2_scale, bn2_bias, bns_scale, bns_bias):
    N, Cin, H, W = x.shape
    HW = H * W
    C = bn1_scale.shape[-1]

    Cin_p = -(-Cin // 8) * 8
    xr = x.reshape(N, Cin, HW).astype(jnp.float32)
    if Cin_p != Cin:
        xr = jnp.pad(xr, ((0, 0), (0, Cin_p - Cin), (0, 0)))

    cols = jnp.broadcast_to(jnp.arange(W).reshape(1, W), (H, W))
    tap_mask = jnp.stack(
        [((cols + dw >= 0) & (cols + dw < W)).reshape(1, HW)
         for dw in (-1, 1)], axis=0).astype(jnp.bfloat16)

    def prep3x3(w_hwio, scale, ci_pad):
        w = w_hwio * scale
        ci = w.shape[2]
        if ci_pad != ci:
            w = jnp.pad(w, ((0, 0), (0, 0), (0, ci_pad - ci), (0, 0)))
        return jnp.transpose(w, (3, 0, 1, 2)).reshape(C, 9 * ci_pad)

    w1 = prep3x3(w1_hwio, bn1_scale, Cin_p)
    w2 = prep3x3(w2_hwio, bn2_scale, C)

    ws = ws_hwio[0, 0] * bns_scale
    if Cin_p != Cin:
        ws = jnp.pad(ws, ((0, Cin_p - Cin), (0, 0)))
    ws = ws.T

    w1s = jnp.concatenate(
        [w1[:, :3 * Cin_p], w1[:, 3 * Cin_p:6 * Cin_p], w1[:, 6 * Cin_p:]],
        axis=0).astype(jnp.bfloat16)
    w2s = jnp.concatenate(
        [w2[:, :3 * C], w2[:, 3 * C:6 * C], w2[:, 6 * C:]], axis=0)
    sc_col = jnp.zeros((3 * C, Cin_p), jnp.float32).at[C:2 * C].set(ws)
    w2e = jnp.concatenate([w2s, sc_col], axis=1).astype(jnp.bfloat16)

    b1 = bn1_bias.reshape(C, 1).astype(jnp.float32)
    b2c = (bn2_bias + bns_bias).reshape(C, 1).astype(jnp.float32)

    def const_spec(shape):
        return pl.BlockSpec(shape, lambda n: (0,) * len(shape))

    BATCH = 2 if N % 2 == 0 else 1
    flops = 2 * N * HW * C * (9 * Cin_p + 9 * C + Cin_p)
    bytes_accessed = (xr.size * 4 + tap_mask.size * 2 + w1s.size * 2 +
                      w2e.size * 2 + 2 * C * 4 + N * C * HW * 4)
    cost = pl.CostEstimate(flops=flops, transcendentals=0,
                           bytes_accessed=bytes_accessed)

    out = pl.pallas_call(
        _make_body(H, W, C, BATCH),
        out_shape=jax.ShapeDtypeStruct((N, C, HW), jnp.float32),
        grid=(N // BATCH,),
        in_specs=[
            pl.BlockSpec((BATCH, Cin_p, HW), lambda n: (n, 0, 0)),
            const_spec((2, 1, HW)),
            const_spec((3 * C, 3 * Cin_p)),
            const_spec((3 * C, 3 * C + Cin_p)),
            const_spec((C, 1)),
            const_spec((C, 1)),
        ],
        out_specs=pl.BlockSpec((BATCH, C, HW), lambda n: (n, 0, 0)),
        compiler_params=pltpu.CompilerParams(
            dimension_semantics=("parallel",)),
        cost_estimate=cost,
    )(xr, tap_mask, w1s, w2e, b1, b2c)

    return out.reshape(N, C, H, W)
